# Initial kernel scaffold; baseline (speedup 1.0000x reference)
import jax
import jax.numpy as jnp
from jax.experimental import pallas as pl


def kernel(x, edge_index, edge_weight, W1, b1, W2, b2):
    raise NotImplementedError("write your pallas kernel here")


# speedup vs baseline: 13.3889x; 13.3889x over previous
import functools

import jax
import jax.numpy as jnp
from jax import lax
from jax.experimental import pallas as pl
from jax.experimental.pallas import tpu as pltpu
from jax.experimental.pallas import tpu_sc as plsc

_NC = 2
_NS = 16
_NW = _NC * _NS
_L = 16


@functools.lru_cache(maxsize=None)
def _make_sc_deg(n, e):
    per_w = e // _NW
    ch = 2000
    nch = per_w // ch
    rem = per_w - nch * ch
    assert rem % _L == 0 and n % _L == 0
    mesh = plsc.VectorSubcoreMesh(core_axis_name="c", subcore_axis_name="s")

    @functools.partial(
        pl.kernel,
        out_type=jax.ShapeDtypeStruct((_NW, n), jnp.float32),
        mesh=mesh,
        compiler_params=pltpu.CompilerParams(needs_layout_passes=False),
        scratch_types=[
            pltpu.VMEM((n,), jnp.float32),
            pltpu.VMEM((ch,), jnp.int32),
            pltpu.VMEM((ch,), jnp.float32),
        ],
    )
    def sc_deg(col_hbm, ew_hbm, out_hbm, hist, col_v, w_v):
        cid = lax.axis_index("c")
        sid = lax.axis_index("s")
        wid = sid * _NC + cid

        @pl.loop(0, n, step=_L)
        def _(i):
            hist[pl.ds(i, _L)] = jnp.zeros((_L,), jnp.float32)

        base_w = wid * per_w

        @pl.loop(0, nch)
        def _(t):
            b = base_w + t * ch
            pltpu.sync_copy(col_hbm.at[pl.ds(b, ch)], col_v)
            pltpu.sync_copy(ew_hbm.at[pl.ds(b, ch)], w_v)

            @pl.loop(0, ch, step=_L)
            def _(g):
                plsc.addupdate_scatter(
                    hist, [col_v[pl.ds(g, _L)]], w_v[pl.ds(g, _L)]
                )

        if rem:
            b = base_w + nch * ch
            pltpu.sync_copy(col_hbm.at[pl.ds(b, rem)], col_v.at[pl.ds(0, rem)])
            pltpu.sync_copy(ew_hbm.at[pl.ds(b, rem)], w_v.at[pl.ds(0, rem)])

            @pl.loop(0, rem, step=_L)
            def _(g):
                plsc.addupdate_scatter(
                    hist, [col_v[pl.ds(g, _L)]], w_v[pl.ds(g, _L)]
                )

        pltpu.sync_copy(hist, out_hbm.at[wid])

    return sc_deg


@functools.lru_cache(maxsize=None)
def _make_sc_scatter(n_pad, d, e, chunk):
    nchunk = e // chunk
    assert nchunk * chunk == e
    per_tile = n_pad // _NS
    ndump = per_tile // chunk
    assert ndump * chunk == per_tile
    jmax = (nchunk + _NW - 1) // _NW
    mesh = plsc.VectorSubcoreMesh(core_axis_name="c", subcore_axis_name="s")

    @functools.partial(
        pl.kernel,
        out_type=jax.ShapeDtypeStruct((_NC, n_pad, d), jnp.float32),
        mesh=mesh,
        scratch_types=[
            pltpu.VMEM((chunk,), jnp.int32),
            pltpu.VMEM((chunk,), jnp.int32),
            pltpu.VMEM((chunk,), jnp.float32),
            pltpu.VMEM((chunk, d), jnp.float32),
            pltpu.VMEM_SHARED((n_pad, d), jnp.float32),
            pltpu.SemaphoreType.DMA,
        ],
    )
    def sc_scatter(xs_hbm, row_hbm, col_hbm, ew_hbm, out_hbm,
                   row_v, col_v, w_v, rows_v, acc, sem):
        cid = lax.axis_index("c")
        sid = lax.axis_index("s")
        wid = sid * _NC + cid

        @pl.loop(0, chunk)
        def _(r):
            for f in range(d // _L):
                rows_v[r, pl.ds(f * _L, _L)] = jnp.zeros((_L,), jnp.float32)

        @pl.loop(0, ndump)
        def _(t):
            pltpu.sync_copy(
                rows_v, acc.at[pl.ds(sid * per_tile + t * chunk, chunk)]
            )

        plsc.subcore_barrier()

        @pl.loop(0, jmax)
        def _(j):
            c = wid + _NW * j

            @pl.when(c < nchunk)
            def _():
                base = c * chunk
                pltpu.sync_copy(row_hbm.at[pl.ds(base, chunk)], row_v)
                pltpu.sync_copy(col_hbm.at[pl.ds(base, chunk)], col_v)
                pltpu.sync_copy(ew_hbm.at[pl.ds(base, chunk)], w_v)
                pltpu.async_copy(xs_hbm.at[row_v], rows_v, sem).wait()

                @pl.loop(0, chunk, step=_L)
                def _(g):
                    wv16 = w_v[pl.ds(g, _L)]
                    for l in range(_L):
                        wvec = jnp.full((_L,), wv16[l], dtype=jnp.float32)
                        ee = g + l
                        for f in range(d // _L):
                            rows_v[ee, pl.ds(f * _L, _L)] = (
                                rows_v[ee, pl.ds(f * _L, _L)] * wvec
                            )

                pltpu.async_copy(rows_v, acc.at[col_v], sem, add=True).wait()

        plsc.subcore_barrier()

        @pl.loop(0, ndump)
        def _(t):
            r0 = sid * per_tile + t * chunk
            pltpu.sync_copy(acc.at[pl.ds(r0, chunk)],
                            out_hbm.at[cid, pl.ds(r0, chunk)])

    return sc_scatter


def _tc_dinv(degp):
    nw, n = degp.shape

    return pl.pallas_call(
        _tc_dinv_body,
        grid=(1,),
        in_specs=[pl.BlockSpec((nw, n), lambda i: (0, 0))],
        out_specs=pl.BlockSpec((n, 1), lambda i: (0, 0)),
        out_shape=jax.ShapeDtypeStruct((n, 1), jnp.float32),
    )(degp)


def _tc_dinv_body(degp_ref, dinv_ref):
    deg = jnp.sum(degp_ref[...], axis=0) + 1.0
    safe = jnp.where(deg > 0, deg, 1.0)
    dinv = jnp.where(deg > 0, lax.rsqrt(safe), 0.0)
    dinv_ref[...] = dinv[:, None]


def _tc_prep(x, dinv, w1):
    n, d_in = x.shape
    d_hid = w1.shape[1]
    r = 1000

    return pl.pallas_call(
        _tc_prep_body,
        grid=(n // r,),
        in_specs=[
            pl.BlockSpec((r, d_in), lambda i: (i, 0)),
            pl.BlockSpec((r, 1), lambda i: (i, 0)),
            pl.BlockSpec((d_in, d_hid), lambda i: (0, 0)),
        ],
        out_specs=pl.BlockSpec((r, d_hid), lambda i: (i, 0)),
        out_shape=jax.ShapeDtypeStruct((n, d_hid), jnp.float32),
    )(x, dinv, w1)


def _tc_prep_body(x_ref, dinv_ref, w_ref, xs_ref):
    xw = jnp.dot(x_ref[...], w_ref[...], preferred_element_type=jnp.float32)
    xs_ref[...] = xw * dinv_ref[...]


def _tc_mid(p, xs, dinv, b, w2):
    n, d = xs.shape
    d_out = w2.shape[1]
    r = 1000

    return pl.pallas_call(
        _tc_mid_body,
        grid=(n // r,),
        in_specs=[
            pl.BlockSpec((1, r, d), lambda i: (0, i, 0)),
            pl.BlockSpec((1, r, d), lambda i: (1, i, 0)),
            pl.BlockSpec((r, d), lambda i: (i, 0)),
            pl.BlockSpec((r, 1), lambda i: (i, 0)),
            pl.BlockSpec((1, d), lambda i: (0, 0)),
            pl.BlockSpec((d, d_out), lambda i: (0, 0)),
        ],
        out_specs=pl.BlockSpec((r, d_out), lambda i: (i, 0)),
        out_shape=jax.ShapeDtypeStruct((n, d_out), jnp.float32),
    )(p, p, xs, dinv, b, w2)


def _tc_mid_body(p0_ref, p1_ref, xs_ref, dinv_ref, b_ref, w_ref, out_ref):
    dinv = dinv_ref[...]
    accum = p0_ref[0] + p1_ref[0] + xs_ref[...]
    h = jnp.maximum(accum * dinv + b_ref[...], 0.0)
    hw = jnp.dot(h, w_ref[...], preferred_element_type=jnp.float32)
    out_ref[...] = hw * dinv


def _tc_fin(p, xs, dinv, b):
    n, d = xs.shape
    r = 1000

    return pl.pallas_call(
        _tc_fin_body,
        grid=(n // r,),
        in_specs=[
            pl.BlockSpec((1, r, d), lambda i: (0, i, 0)),
            pl.BlockSpec((1, r, d), lambda i: (1, i, 0)),
            pl.BlockSpec((r, d), lambda i: (i, 0)),
            pl.BlockSpec((r, 1), lambda i: (i, 0)),
            pl.BlockSpec((1, d), lambda i: (0, 0)),
        ],
        out_specs=pl.BlockSpec((r, d), lambda i: (i, 0)),
        out_shape=jax.ShapeDtypeStruct((n, d), jnp.float32),
    )(p, p, xs, dinv, b)


def _tc_fin_body(p0_ref, p1_ref, xs_ref, dinv_ref, b_ref, out_ref):
    accum = p0_ref[0] + p1_ref[0] + xs_ref[...]
    out_ref[...] = accum * dinv_ref[...] + b_ref[...]


def kernel(x, edge_index, edge_weight, W1, b1, W2, b2):
    n, _ = x.shape
    e = edge_weight.shape[0]
    d = W1.shape[1]
    edge_index = edge_index.astype(jnp.int32)
    row = edge_index[0]
    col = edge_index[1]
    chunk = 128
    n_pad = ((n + _NS * chunk - 1) // (_NS * chunk)) * (_NS * chunk)

    sc_deg = _make_sc_deg(n, e)
    sc_scatter = _make_sc_scatter(n_pad, d, e, chunk)

    degp = sc_deg(col, edge_weight)
    dinv = _tc_dinv(degp)
    xs1 = _tc_prep(x, dinv, W1)
    p1 = sc_scatter(xs1, row, col, edge_weight)
    xs2 = _tc_mid(p1, xs1, dinv, b1.reshape(1, -1), W2)
    p2 = sc_scatter(xs2, row, col, edge_weight)
    out = _tc_fin(p2, xs2, dinv, b2.reshape(1, -1))
    return out
